# Initial kernel scaffold; baseline (speedup 1.0000x reference)
import functools

import jax
import jax.numpy as jnp
from jax import lax
from jax.experimental import pallas as pl
from jax.experimental.pallas import tpu as pltpu
from jax.experimental.pallas import tpu_sc as plsc

F32 = jnp.float32
I32 = jnp.int32

N = 10000
E = 320000
D_IN = 128
D_HID = 256
D_OUT = 128

NC, NS = 2, 16
NW = NC * NS
K = 128
WPW = 80
EP = NW * WPW * K
ROWS = EP // K
TROWS = 10240
RPT = TROWS // NS
NBUF = 4

_mesh = plsc.VectorSubcoreMesh(core_axis_name="c", subcore_axis_name="s")


def _zero_vmem_rows(buf, nrows):
    def zrow(i, carry):
        for l in range(8):
            buf[i, pl.ds(l * 16, 16)] = jnp.zeros((16,), F32)
        return carry
    lax.fori_loop(0, nrows, zrow, 0)


def _deg_body(dst_hbm, deg_out, deg_sh, dst_v, ones_v, wb_v, dsem):
    cid = lax.axis_index("c")
    sid = lax.axis_index("s")
    wid = cid * NS + sid

    for l in range(8):
        ones_v[pl.ds(l * 16, 16)] = jnp.full((16,), 1.0, F32)

    def zr(i, carry):
        wb_v[pl.ds(i * 16, 16)] = jnp.zeros((16,), F32)
        return carry
    lax.fori_loop(0, RPT // 16, zr, 0)

    pltpu.sync_copy(wb_v, deg_sh.at[pl.ds(sid * RPT, RPT)])
    plsc.subcore_barrier()

    pltpu.sync_copy(dst_hbm.at[pl.ds(wid * WPW, WPW)], dst_v)

    def step(g, carry):
        for b in range(NBUF):
            j = g * NBUF + b
            pltpu.async_copy(ones_v, deg_sh.at[dst_v.at[j]], dsem, add=True)
        for b in range(NBUF):
            j = g * NBUF + b
            pltpu.make_async_copy(ones_v, deg_sh.at[dst_v.at[j]], dsem).wait()
        return carry
    lax.fori_loop(0, WPW // NBUF, step, 0)

    plsc.subcore_barrier()
    pltpu.sync_copy(deg_sh.at[pl.ds(sid * RPT, RPT)], wb_v)
    pltpu.sync_copy(wb_v, deg_out.at[cid, pl.ds(sid * RPT, RPT)])


_deg_kernel = pl.kernel(
    _deg_body,
    out_type=jax.ShapeDtypeStruct((NC, TROWS), F32),
    mesh=_mesh,
    scratch_types=[
        pltpu.VMEM_SHARED((TROWS,), F32),
        pltpu.VMEM((WPW, K), I32),
        pltpu.VMEM((K,), F32),
        pltpu.VMEM((RPT,), F32),
        pltpu.SemaphoreType.DMA,
    ],
)


def _conv_body(nchunks, *refs):
    src_hbm, dst_hbm = refs[0], refs[1]
    y_hbms = refs[2:2 + nchunks]
    out_hbms = refs[2 + nchunks:2 + 2 * nchunks]
    sc = refs[2 + 2 * nchunks:]
    acc_sh = sc[0]
    src_v, dst_v = sc[1], sc[2]
    rbufs = sc[3:3 + NBUF]
    zbuf = sc[3 + NBUF]
    gsems = sc[4 + NBUF:4 + 2 * NBUF]
    ssems = sc[4 + 2 * NBUF:4 + 3 * NBUF]

    cid = lax.axis_index("c")
    sid = lax.axis_index("s")
    wid = cid * NS + sid

    pltpu.sync_copy(src_hbm.at[pl.ds(wid * WPW, WPW)], src_v)
    pltpu.sync_copy(dst_hbm.at[pl.ds(wid * WPW, WPW)], dst_v)

    for c in range(nchunks):
        y_hbm = y_hbms[c]
        out_hbm = out_hbms[c]

        _zero_vmem_rows(zbuf, K)
        for q in range(RPT // K):
            pltpu.sync_copy(zbuf, acc_sh.at[pl.ds(sid * RPT + q * K, K), :])
        plsc.subcore_barrier()

        for b in range(NBUF):
            pltpu.async_copy(y_hbm.at[src_v.at[b]], rbufs[b], gsems[b])

        def step(g, carry):
            for b in range(NBUF):
                j = g * NBUF + b
                pltpu.make_async_copy(
                    y_hbm.at[src_v.at[j]], rbufs[b], gsems[b]).wait()
                pltpu.async_copy(
                    rbufs[b], acc_sh.at[dst_v.at[j]], ssems[b], add=True)
            for b in range(NBUF):
                j = g * NBUF + b
                pltpu.make_async_copy(
                    rbufs[b], acc_sh.at[dst_v.at[j]], ssems[b]).wait()

                @pl.when(g < WPW // NBUF - 1)
                def _issue_next():
                    pltpu.async_copy(
                        y_hbm.at[src_v.at[j + NBUF]], rbufs[b], gsems[b])
            return carry
        lax.fori_loop(0, WPW // NBUF, step, 0)

        plsc.subcore_barrier()
        for q in range(RPT // K):
            r0 = sid * RPT + q * K
            pltpu.sync_copy(acc_sh.at[pl.ds(r0, K), :], zbuf)
            pltpu.sync_copy(zbuf, out_hbm.at[cid, pl.ds(r0, K), :])


def _make_conv(nchunks):
    return pl.kernel(
        functools.partial(_conv_body, nchunks),
        out_type=[jax.ShapeDtypeStruct((NC, TROWS, 128), F32)] * nchunks,
        mesh=_mesh,
        scratch_types=(
            [pltpu.VMEM_SHARED((TROWS, 128), F32),
             pltpu.VMEM((WPW, K), I32),
             pltpu.VMEM((WPW, K), I32)]
            + [pltpu.VMEM((K, 128), F32) for _ in range(NBUF)]
            + [pltpu.VMEM((K, 128), F32)]
            + [pltpu.SemaphoreType.DMA for _ in range(2 * NBUF)]
        ),
    )


_conv1_kernel = _make_conv(2)
_conv2_kernel = _make_conv(1)


RB = 400
GRID = N // RB


def _dinv_block(d0, d1):
    return lax.rsqrt(d0 + d1 + 1.0)


def _tc1_body(d0_ref, d1_ref, x_ref, w1_ref, y1a_ref, y1b_ref):
    dinv = _dinv_block(d0_ref[...], d1_ref[...])
    xw = jnp.dot(x_ref[...], w1_ref[...], preferred_element_type=F32)
    y = xw * dinv
    y1a_ref[...] = y[:, :128]
    y1b_ref[...] = y[:, 128:]


def _tc2_body(d0_ref, d1_ref, s1a_ref, s1b_ref, y1a_ref, y1b_ref,
              b1_ref, w2_ref, y2_ref):
    dinv = _dinv_block(d0_ref[...], d1_ref[...])
    b1 = b1_ref[...]
    h0 = jnp.maximum(dinv * (s1a_ref[0] + s1a_ref[1] + y1a_ref[...])
                     + b1[:, :128], 0.0)
    h1 = jnp.maximum(dinv * (s1b_ref[0] + s1b_ref[1] + y1b_ref[...])
                     + b1[:, 128:], 0.0)
    h = jnp.concatenate([h0, h1], axis=1)
    y2_ref[...] = jnp.dot(h, w2_ref[...], preferred_element_type=F32) * dinv


def _tc3_body(d0_ref, d1_ref, s2_ref, y2_ref, b2_ref,
              f1w_ref, f1b_ref, f2w_ref, f2b_ref, o_ref):
    dinv = _dinv_block(d0_ref[...], d1_ref[...])
    z = jnp.maximum(dinv * (s2_ref[0] + s2_ref[1] + y2_ref[...])
                    + b2_ref[...], 0.0)
    t = jnp.dot(z, f1w_ref[...], preferred_element_type=F32) + f1b_ref[...]
    p = jnp.where(t > 0.0, t, jnp.expm1(t))
    o_ref[...] = jnp.dot(p, f2w_ref[...], preferred_element_type=F32) \
        + f2b_ref[...]


def _row_spec(shape):
    nd = len(shape)
    if nd == 2:
        return pl.BlockSpec((RB, shape[1]), lambda i: (i, 0))
    return pl.BlockSpec((shape[0], RB, shape[2]), lambda i: (0, i, 0))


def _full_spec(shape):
    return pl.BlockSpec(shape, lambda i: (0,) * len(shape))


def _tc_call(body, ins_row, ins_full, out_shapes):
    in_specs = [_row_spec(a.shape) for a in ins_row] + \
               [_full_spec(a.shape) for a in ins_full]
    out_specs = [_row_spec(s.shape) for s in out_shapes]
    outs = pl.pallas_call(
        body,
        grid=(GRID,),
        in_specs=in_specs,
        out_specs=out_specs if len(out_shapes) > 1 else out_specs[0],
        out_shape=out_shapes if len(out_shapes) > 1 else out_shapes[0],
    )(*ins_row, *ins_full)
    return outs


def kernel(x, edge_index, W1, b1, W2, b2, fc1_W, fc1_b, fc2_W, fc2_b):
    ei = edge_index.astype(I32)
    src, dst = ei[0], ei[1]
    pad = EP - E
    fk = jnp.arange(pad, dtype=I32)
    fake_src = (fk * 131) % N
    fake_dst = N + (fk % 128)
    srcp = jnp.concatenate([src, fake_src]).reshape(ROWS, K)
    dstp = jnp.concatenate([dst, fake_dst]).reshape(ROWS, K)

    degp = _deg_kernel(dstp)
    d0 = degp[0].reshape(TROWS, 1)
    d1 = degp[1].reshape(TROWS, 1)

    sds = jax.ShapeDtypeStruct
    y1a, y1b = _tc_call(
        _tc1_body, [d0, d1, x], [W1],
        [sds((N, 128), F32), sds((N, 128), F32)])

    s1a, s1b = _conv1_kernel(srcp, dstp, y1a, y1b)

    y2 = _tc_call(
        _tc2_body, [d0, d1, s1a, s1b, y1a, y1b],
        [b1.reshape(1, D_HID), W2],
        [sds((N, 128), F32)])

    (s2,) = _conv2_kernel(srcp, dstp, y2)

    out = _tc_call(
        _tc3_body, [d0, d1, s2, y2],
        [b2.reshape(1, 128), fc1_W, fc1_b.reshape(1, 128),
         fc2_W, fc2_b.reshape(1, 128)],
        [sds((N, D_OUT), F32)])
    return out


# speedup vs baseline: 20.5592x; 20.5592x over previous
import functools

import jax
import jax.numpy as jnp
from jax import lax
from jax.experimental import pallas as pl
from jax.experimental.pallas import tpu as pltpu
from jax.experimental.pallas import tpu_sc as plsc

F32 = jnp.float32
I32 = jnp.int32

N = 10000
E = 320000
D_HID = 256
D_OUT = 128
CW = 64

NC, NS = 2, 16
NW = NC * NS
K = 128
WPW = 80
EP = NW * WPW * K
ROWS = EP // K
TROWS = 10240
RPT = TROWS // NS
NBUF = 4

_mesh = plsc.VectorSubcoreMesh(core_axis_name="c", subcore_axis_name="s")


def _deg_body(idx_hbm, deg_out, deg_sh, idx_v, ones_v, wb_v, dsem):
    cid = lax.axis_index("c")
    sid = lax.axis_index("s")
    wid = cid * NS + sid

    for l in range(8):
        ones_v[pl.ds(l * 16, 16)] = jnp.full((16,), 1.0, F32)

    def zr(i, carry):
        wb_v[pl.ds(i * 16, 16)] = jnp.zeros((16,), F32)
        return carry
    lax.fori_loop(0, RPT // 16, zr, 0)

    pltpu.sync_copy(wb_v, deg_sh.at[pl.ds(sid * RPT, RPT)])
    plsc.subcore_barrier()

    pltpu.sync_copy(idx_hbm.at[pl.ds(wid * WPW, WPW)], idx_v)

    def step(g, carry):
        for b in range(NBUF):
            j = g * NBUF + b
            pltpu.async_copy(ones_v, deg_sh.at[idx_v.at[j, 1]], dsem,
                             add=True)
        for b in range(NBUF):
            j = g * NBUF + b
            pltpu.make_async_copy(ones_v, deg_sh.at[idx_v.at[j, 1]],
                                  dsem).wait()
        return carry
    lax.fori_loop(0, WPW // NBUF, step, 0)

    plsc.subcore_barrier()
    pltpu.sync_copy(deg_sh.at[pl.ds(sid * RPT, RPT)], wb_v)
    pltpu.sync_copy(wb_v, deg_out.at[cid, pl.ds(sid * RPT, RPT)])


_deg_kernel = pl.kernel(
    _deg_body,
    out_type=jax.ShapeDtypeStruct((NC, TROWS), F32),
    mesh=_mesh,
    scratch_types=[
        pltpu.VMEM_SHARED((TROWS,), F32),
        pltpu.VMEM((WPW, 2, K), I32),
        pltpu.VMEM((K,), F32),
        pltpu.VMEM((RPT,), F32),
        pltpu.SemaphoreType.DMA,
    ],
)


def _conv_body(nchunks, *refs):
    idx_hbm = refs[0]
    y_hbms = refs[1:1 + nchunks]
    out_hbms = refs[1 + nchunks:1 + 2 * nchunks]
    sc = refs[1 + 2 * nchunks:]
    acc_sh = sc[0]
    idx_v = sc[1]
    rbufs = sc[2:2 + NBUF]
    zbuf = sc[2 + NBUF]
    gsems = sc[3 + NBUF:3 + 2 * NBUF]
    ssems = sc[3 + 2 * NBUF:3 + 3 * NBUF]

    cid = lax.axis_index("c")
    sid = lax.axis_index("s")
    wid = cid * NS + sid

    pltpu.sync_copy(idx_hbm.at[pl.ds(wid * WPW, WPW)], idx_v)

    for c in range(nchunks):
        y_hbm = y_hbms[c]
        out_hbm = out_hbms[c]

        def zrow(i, carry):
            for l in range(CW // 16):
                zbuf[i, pl.ds(l * 16, 16)] = jnp.zeros((16,), F32)
            return carry
        lax.fori_loop(0, K, zrow, 0)
        for q in range(RPT // K):
            pltpu.sync_copy(zbuf, acc_sh.at[pl.ds(sid * RPT + q * K, K), :])
        plsc.subcore_barrier()

        for b in range(NBUF):
            pltpu.async_copy(y_hbm.at[idx_v.at[b, 0]], rbufs[b], gsems[b])

        def step(g, carry):
            for b in range(NBUF):
                j = g * NBUF + b
                pltpu.make_async_copy(
                    y_hbm.at[idx_v.at[j, 0]], rbufs[b], gsems[b]).wait()
                pltpu.async_copy(
                    rbufs[b], acc_sh.at[idx_v.at[j, 1]], ssems[b], add=True)
            for b in range(NBUF):
                j = g * NBUF + b
                pltpu.make_async_copy(
                    rbufs[b], acc_sh.at[idx_v.at[j, 1]], ssems[b]).wait()

                @pl.when(g < WPW // NBUF - 1)
                def _issue_next():
                    pltpu.async_copy(
                        y_hbm.at[idx_v.at[j + NBUF, 0]], rbufs[b], gsems[b])
            return carry
        lax.fori_loop(0, WPW // NBUF, step, 0)

        plsc.subcore_barrier()
        for q in range(RPT // K):
            r0 = sid * RPT + q * K
            pltpu.sync_copy(acc_sh.at[pl.ds(r0, K), :], zbuf)
            pltpu.sync_copy(zbuf, out_hbm.at[cid, pl.ds(r0, K), :])


def _make_conv(nchunks):
    return pl.kernel(
        functools.partial(_conv_body, nchunks),
        out_type=[jax.ShapeDtypeStruct((NC, TROWS, CW), F32)] * nchunks,
        mesh=_mesh,
        compiler_params=pltpu.CompilerParams(use_tc_tiling_on_sc=False),
        scratch_types=(
            [pltpu.VMEM_SHARED((TROWS, CW), F32),
             pltpu.VMEM((WPW, 2, K), I32)]
            + [pltpu.VMEM((K, CW), F32) for _ in range(NBUF)]
            + [pltpu.VMEM((K, CW), F32)]
            + [pltpu.SemaphoreType.DMA for _ in range(2 * NBUF)]
        ),
    )


_conv1_kernel = _make_conv(4)
_conv2_kernel = _make_conv(2)


RB = 400
GRID = N // RB


def _dinv_block(d0, d1):
    return lax.rsqrt(d0 + d1 + 1.0)


def _tc1_body(d0_ref, d1_ref, x_ref, w1_ref, *y_refs):
    dinv = _dinv_block(d0_ref[...], d1_ref[...])
    xw = jnp.dot(x_ref[...], w1_ref[...], preferred_element_type=F32)
    y = xw * dinv
    for c in range(4):
        y_refs[c][...] = y[:, c * CW:(c + 1) * CW]


def _tc2_body(d0_ref, d1_ref, s10, s11, s12, s13, y10, y11, y12, y13,
              b1_ref, w2_ref, y20_ref, y21_ref):
    dinv = _dinv_block(d0_ref[...], d1_ref[...])
    b1 = b1_ref[...]
    hs = []
    for c, (s, y) in enumerate(((s10, y10), (s11, y11), (s12, y12),
                                (s13, y13))):
        hs.append(jnp.maximum(
            dinv * (s[0] + s[1] + y[...]) + b1[:, c * CW:(c + 1) * CW], 0.0))
    h = jnp.concatenate(hs, axis=1)
    y2 = jnp.dot(h, w2_ref[...], preferred_element_type=F32) * dinv
    y20_ref[...] = y2[:, :CW]
    y21_ref[...] = y2[:, CW:]


def _tc3_body(d0_ref, d1_ref, s20, s21, y20, y21, b2_ref,
              f1w_ref, f1b_ref, f2w_ref, f2b_ref, o_ref):
    dinv = _dinv_block(d0_ref[...], d1_ref[...])
    zs = []
    for c, (s, y) in enumerate(((s20, y20), (s21, y21))):
        zs.append(jnp.maximum(
            dinv * (s[0] + s[1] + y[...])
            + b2_ref[..., c * CW:(c + 1) * CW], 0.0))
    z = jnp.concatenate(zs, axis=1)
    t = jnp.dot(z, f1w_ref[...], preferred_element_type=F32) + f1b_ref[...]
    p = jnp.where(t > 0.0, t, jnp.exp(t) - 1.0)
    o_ref[...] = jnp.dot(p, f2w_ref[...], preferred_element_type=F32) \
        + f2b_ref[...]


def _row_spec(shape):
    nd = len(shape)
    if nd == 2:
        return pl.BlockSpec((RB, shape[1]), lambda i: (i, 0))
    return pl.BlockSpec((shape[0], RB, shape[2]), lambda i: (0, i, 0))


def _full_spec(shape):
    return pl.BlockSpec(shape, lambda i: (0,) * len(shape))


def _tc_call(body, ins_row, ins_full, out_shapes):
    in_specs = [_row_spec(a.shape) for a in ins_row] + \
               [_full_spec(a.shape) for a in ins_full]
    out_specs = [_row_spec(s.shape) for s in out_shapes]
    outs = pl.pallas_call(
        body,
        grid=(GRID,),
        in_specs=in_specs,
        out_specs=out_specs if len(out_shapes) > 1 else out_specs[0],
        out_shape=out_shapes if len(out_shapes) > 1 else out_shapes[0],
    )(*ins_row, *ins_full)
    return outs


def kernel(x, edge_index, W1, b1, W2, b2, fc1_W, fc1_b, fc2_W, fc2_b):
    ei = edge_index.astype(I32)
    src, dst = ei[0], ei[1]
    pad = EP - E
    fk = jnp.arange(pad, dtype=I32)
    fake_src = (fk * 131) % N
    fake_dst = N + (fk % 128)
    srcp = jnp.concatenate([src, fake_src]).reshape(ROWS, K)
    dstp = jnp.concatenate([dst, fake_dst]).reshape(ROWS, K)
    idxp = jnp.stack([srcp, dstp], axis=1)

    degp = _deg_kernel(idxp)
    d0 = degp[0].reshape(TROWS, 1)
    d1 = degp[1].reshape(TROWS, 1)

    sds = jax.ShapeDtypeStruct
    y1 = _tc_call(
        _tc1_body, [d0, d1, x], [W1],
        [sds((N, CW), F32)] * 4)

    s1 = _conv1_kernel(idxp, *y1)

    y2 = _tc_call(
        _tc2_body, [d0, d1] + list(s1) + list(y1),
        [b1.reshape(1, D_HID), W2],
        [sds((N, CW), F32)] * 2)

    s2 = _conv2_kernel(idxp, *y2)

    out = _tc_call(
        _tc3_body, [d0, d1] + list(s2) + list(y2),
        [b2.reshape(1, 128), fc1_W, fc1_b.reshape(1, 128),
         fc2_W, fc2_b.reshape(1, 128)],
        [sds((N, D_OUT), F32)])
    return out


# speedup vs baseline: 21.9945x; 1.0698x over previous
import functools

import jax
import jax.numpy as jnp
from jax import lax
from jax.experimental import pallas as pl
from jax.experimental.pallas import tpu as pltpu
from jax.experimental.pallas import tpu_sc as plsc

F32 = jnp.float32
I32 = jnp.int32

N = 10000
E = 320000
D_HID = 256
D_OUT = 128
CW = 64

NC, NS = 2, 16
NW = NC * NS
K = 128
WPW = 80
EP = NW * WPW * K
ROWS = EP // K
TROWS = 10240
RPT = TROWS // NS
NBUF = 4

_mesh = plsc.VectorSubcoreMesh(core_axis_name="c", subcore_axis_name="s")


def _deg_body(dst_hbm, deg0_out, deg1_out, deg_sh, dst_v, ones_v, wb_v,
              dsem):
    cid = lax.axis_index("c")
    sid = lax.axis_index("s")
    wid = cid * NS + sid

    for l in range(8):
        ones_v[pl.ds(l * 16, 16)] = jnp.full((16,), 1.0, F32)

    def zr(i, carry):
        wb_v[pl.ds(i * 16, 16)] = jnp.zeros((16,), F32)
        return carry
    lax.fori_loop(0, RPT // 16, zr, 0)

    pltpu.sync_copy(wb_v, deg_sh.at[pl.ds(sid * RPT, RPT)])
    plsc.subcore_barrier()

    pltpu.sync_copy(dst_hbm.at[pl.ds(wid * WPW, WPW)], dst_v)

    def step(g, carry):
        for b in range(NBUF):
            j = g * NBUF + b
            pltpu.async_copy(ones_v, deg_sh.at[dst_v.at[j]], dsem, add=True)
        for b in range(NBUF):
            j = g * NBUF + b
            pltpu.make_async_copy(ones_v, deg_sh.at[dst_v.at[j]],
                                  dsem).wait()
        return carry
    lax.fori_loop(0, WPW // NBUF, step, 0)

    plsc.subcore_barrier()
    pltpu.sync_copy(deg_sh.at[pl.ds(sid * RPT, RPT)], wb_v)

    @pl.when(cid == 0)
    def _wb0():
        pltpu.sync_copy(wb_v, deg0_out.at[pl.ds(sid * RPT, RPT)])

    @pl.when(cid == 1)
    def _wb1():
        pltpu.sync_copy(wb_v, deg1_out.at[pl.ds(sid * RPT, RPT)])


_deg_kernel = pl.kernel(
    _deg_body,
    out_type=[jax.ShapeDtypeStruct((TROWS,), F32)] * 2,
    mesh=_mesh,
    scratch_types=[
        pltpu.VMEM_SHARED((TROWS,), F32),
        pltpu.VMEM((WPW, K), I32),
        pltpu.VMEM((K,), F32),
        pltpu.VMEM((RPT,), F32),
        pltpu.SemaphoreType.DMA,
    ],
)


def _conv_body(src_hbm, dst_hbm, y_hbm, out_hbm, src_v, dst_v, acc_sh,
               *rest):
    rbufs = rest[:NBUF]
    zbuf = rest[NBUF]
    gsems = rest[1 + NBUF:1 + 2 * NBUF]
    ssems = rest[1 + 2 * NBUF:1 + 3 * NBUF]

    cid = lax.axis_index("c")
    sid = lax.axis_index("s")
    wid = cid * NS + sid

    pltpu.sync_copy(src_hbm.at[pl.ds(wid * WPW, WPW)], src_v)
    pltpu.sync_copy(dst_hbm.at[pl.ds(wid * WPW, WPW)], dst_v)

    def zrow(i, carry):
        for l in range(CW // 16):
            zbuf[i, pl.ds(l * 16, 16)] = jnp.zeros((16,), F32)
        return carry
    lax.fori_loop(0, K, zrow, 0)
    for q in range(RPT // K):
        pltpu.sync_copy(zbuf, acc_sh.at[pl.ds(sid * RPT + q * K, K), :])
    plsc.subcore_barrier()

    for b in range(NBUF):
        pltpu.async_copy(y_hbm.at[src_v.at[b]], rbufs[b], gsems[b])

    def step(g, carry):
        for b in range(NBUF):
            j = g * NBUF + b
            pltpu.make_async_copy(
                y_hbm.at[src_v.at[j]], rbufs[b], gsems[b]).wait()
            pltpu.async_copy(
                rbufs[b], acc_sh.at[dst_v.at[j]], ssems[b], add=True)
        for b in range(NBUF):
            j = g * NBUF + b
            pltpu.make_async_copy(
                rbufs[b], acc_sh.at[dst_v.at[j]], ssems[b]).wait()

            @pl.when(g < WPW // NBUF - 1)
            def _issue_next():
                pltpu.async_copy(
                    y_hbm.at[src_v.at[j + NBUF]], rbufs[b], gsems[b])
        return carry
    lax.fori_loop(0, WPW // NBUF, step, 0)

    plsc.subcore_barrier()
    for q in range(RPT // K):
        r0 = sid * RPT + q * K
        pltpu.sync_copy(acc_sh.at[pl.ds(r0, K), :], zbuf)
        pltpu.sync_copy(zbuf, out_hbm.at[cid, pl.ds(r0, K), :])


_conv_kernel = pl.kernel(
    _conv_body,
    out_type=jax.ShapeDtypeStruct((NC, TROWS, CW), F32),
    mesh=_mesh,
    compiler_params=pltpu.CompilerParams(use_tc_tiling_on_sc=False),
    scratch_types=(
        [pltpu.VMEM((WPW, K), I32),
         pltpu.VMEM((WPW, K), I32),
         pltpu.VMEM_SHARED((TROWS, CW), F32)]
        + [pltpu.VMEM((K, CW), F32) for _ in range(NBUF)]
        + [pltpu.VMEM((K, CW), F32)]
        + [pltpu.SemaphoreType.DMA for _ in range(2 * NBUF)]
    ),
)


RB = 1000
GRID = N // RB


def _dinv_block(d0_ref, d1_ref):
    return lax.rsqrt(d0_ref[...] + d1_ref[...] + 1.0)


def _tc1_body(x_ref, d0_ref, d1_ref, w1_ref, *y_refs):
    dinv = _dinv_block(d0_ref, d1_ref)
    xw = jnp.dot(x_ref[...], w1_ref[...], preferred_element_type=F32)
    y = xw * dinv
    for c in range(4):
        y_refs[c][...] = y[:, c * CW:(c + 1) * CW]


def _tc2_body(s10, s11, s12, s13, y10, y11, y12, y13,
              d0_ref, d1_ref, b1_ref, w2_ref, y20_ref, y21_ref):
    dinv = _dinv_block(d0_ref, d1_ref)
    b1 = b1_ref[...]
    hs = []
    for c, (s, y) in enumerate(((s10, y10), (s11, y11), (s12, y12),
                                (s13, y13))):
        hs.append(jnp.maximum(
            dinv * (s[0] + s[1] + y[...]) + b1[:, c * CW:(c + 1) * CW], 0.0))
    h = jnp.concatenate(hs, axis=1)
    y2 = jnp.dot(h, w2_ref[...], preferred_element_type=F32) * dinv
    y20_ref[...] = y2[:, :CW]
    y21_ref[...] = y2[:, CW:]


def _tc3_body(s20, s21, y20, y21, d0_ref, d1_ref, b2_ref,
              f1w_ref, f1b_ref, f2w_ref, f2b_ref, o_ref):
    dinv = _dinv_block(d0_ref, d1_ref)
    zs = []
    for c, (s, y) in enumerate(((s20, y20), (s21, y21))):
        zs.append(jnp.maximum(
            dinv * (s[0] + s[1] + y[...])
            + b2_ref[..., c * CW:(c + 1) * CW], 0.0))
    z = jnp.concatenate(zs, axis=1)
    t = jnp.dot(z, f1w_ref[...], preferred_element_type=F32) + f1b_ref[...]
    p = jnp.where(t > 0.0, t, jnp.exp(t) - 1.0)
    o_ref[...] = jnp.dot(p, f2w_ref[...], preferred_element_type=F32) \
        + f2b_ref[...]


def _row_spec(shape):
    nd = len(shape)
    if nd == 1:
        return pl.BlockSpec((RB,), lambda i: (i,))
    if nd == 2:
        return pl.BlockSpec((RB, shape[1]), lambda i: (i, 0))
    return pl.BlockSpec((shape[0], RB, shape[2]), lambda i: (0, i, 0))


def _full_spec(shape):
    return pl.BlockSpec(shape, lambda i: (0,) * len(shape))


def _tc_call(body, ins_row, ins_full, out_shapes):
    in_specs = [_row_spec(a.shape) for a in ins_row] + \
               [_full_spec(a.shape) for a in ins_full]
    out_specs = [_row_spec(s.shape) for s in out_shapes]
    outs = pl.pallas_call(
        body,
        grid=(GRID,),
        in_specs=in_specs,
        out_specs=out_specs if len(out_shapes) > 1 else out_specs[0],
        out_shape=out_shapes if len(out_shapes) > 1 else out_shapes[0],
    )(*ins_row, *ins_full)
    return outs


def kernel(x, edge_index, W1, b1, W2, b2, fc1_W, fc1_b, fc2_W, fc2_b):
    ei = edge_index.astype(I32)
    src, dst = ei[0], ei[1]
    pad = EP - E
    fk = jnp.arange(pad, dtype=I32)
    fake_src = (fk * 131) % N
    fake_dst = N + (fk % 128)
    srcp = jnp.concatenate([src, fake_src]).reshape(ROWS, K)
    dstp = jnp.concatenate([dst, fake_dst]).reshape(ROWS, K)

    d0, d1 = _deg_kernel(dstp)
    d0 = d0.reshape(TROWS, 1)
    d1 = d1.reshape(TROWS, 1)

    sds = jax.ShapeDtypeStruct
    y1 = _tc_call(
        _tc1_body, [x, d0, d1], [W1],
        [sds((N, CW), F32)] * 4)

    s1 = [_conv_kernel(srcp, dstp, yc) for yc in y1]

    y2 = _tc_call(
        _tc2_body, list(s1) + list(y1) + [d0, d1],
        [b1.reshape(1, D_HID), W2],
        [sds((N, CW), F32)] * 2)

    s2 = [_conv_kernel(srcp, dstp, yc) for yc in y2]

    out = _tc_call(
        _tc3_body, list(s2) + list(y2) + [d0, d1],
        [b2.reshape(1, 128), fc1_W, fc1_b.reshape(1, 128),
         fc2_W, fc2_b.reshape(1, 128)],
        [sds((N, D_OUT), F32)])
    return out


# speedup vs baseline: 23.7460x; 1.0796x over previous
import functools

import jax
import jax.numpy as jnp
from jax import lax
from jax.experimental import pallas as pl
from jax.experimental.pallas import tpu as pltpu
from jax.experimental.pallas import tpu_sc as plsc

F32 = jnp.float32
I32 = jnp.int32

N = 10000
E = 320000
D_HID = 256
D_OUT = 128
CW = 64

NC, NS = 2, 16
NW = NC * NS
K = 128
WPW = 80
EP = NW * WPW * K
ROWS = EP // K
TROWS = 10240
RPT = TROWS // NS
NBUF = 8

_mesh = plsc.VectorSubcoreMesh(core_axis_name="c", subcore_axis_name="s")


def _deg_body(dst_hbm, deg0_out, deg1_out, deg_sh, dst_v, ones_v, wb_v,
              dsem):
    cid = lax.axis_index("c")
    sid = lax.axis_index("s")
    wid = cid * NS + sid

    for l in range(8):
        ones_v[pl.ds(l * 16, 16)] = jnp.full((16,), 1.0, F32)

    def zr(i, carry):
        wb_v[pl.ds(i * 16, 16)] = jnp.zeros((16,), F32)
        return carry
    lax.fori_loop(0, RPT // 16, zr, 0)

    pltpu.sync_copy(wb_v, deg_sh.at[pl.ds(sid * RPT, RPT)])
    plsc.subcore_barrier()

    pltpu.sync_copy(dst_hbm.at[pl.ds(wid * WPW, WPW)], dst_v)

    def step(g, carry):
        for b in range(NBUF):
            j = g * NBUF + b
            pltpu.async_copy(ones_v, deg_sh.at[dst_v.at[j]], dsem, add=True)
        for b in range(NBUF):
            j = g * NBUF + b
            pltpu.make_async_copy(ones_v, deg_sh.at[dst_v.at[j]],
                                  dsem).wait()
        return carry
    lax.fori_loop(0, WPW // NBUF, step, 0)

    plsc.subcore_barrier()
    pltpu.sync_copy(deg_sh.at[pl.ds(sid * RPT, RPT)], wb_v)

    @pl.when(cid == 0)
    def _wb0():
        pltpu.sync_copy(wb_v, deg0_out.at[pl.ds(sid * RPT, RPT)])

    @pl.when(cid == 1)
    def _wb1():
        pltpu.sync_copy(wb_v, deg1_out.at[pl.ds(sid * RPT, RPT)])


_deg_kernel = pl.kernel(
    _deg_body,
    out_type=[jax.ShapeDtypeStruct((TROWS,), F32)] * 2,
    mesh=_mesh,
    scratch_types=[
        pltpu.VMEM_SHARED((TROWS,), F32),
        pltpu.VMEM((WPW, K), I32),
        pltpu.VMEM((K,), F32),
        pltpu.VMEM((RPT,), F32),
        pltpu.SemaphoreType.DMA,
    ],
)


def _conv_body(src_hbm, dst_hbm, y_hbm, out_hbm, src_v, dst_v, acc_sh,
               rbuf, gsem, ssem):
    cid = lax.axis_index("c")
    sid = lax.axis_index("s")
    wid = cid * NS + sid

    cp_s = pltpu.async_copy(src_hbm.at[pl.ds(wid * WPW, WPW)], src_v,
                            gsem.at[0])
    cp_d = pltpu.async_copy(dst_hbm.at[pl.ds(wid * WPW, WPW)], dst_v,
                            gsem.at[1])

    def zrow(i, carry):
        for l in range(CW // 16):
            rbuf[0, i, pl.ds(l * 16, 16)] = jnp.zeros((16,), F32)
        return carry
    lax.fori_loop(0, K, zrow, 0)

    def zcp(q, carry):
        pltpu.async_copy(rbuf.at[0],
                         acc_sh.at[pl.ds(sid * RPT + q * K, K), :],
                         ssem.at[0])
        return carry
    lax.fori_loop(0, RPT // K, zcp, 0)

    def zdr(q, carry):
        pltpu.make_async_copy(
            rbuf.at[0], acc_sh.at[pl.ds(sid * RPT, K), :], ssem.at[0]).wait()
        return carry
    lax.fori_loop(0, RPT // K, zdr, 0)
    cp_s.wait()
    cp_d.wait()
    plsc.subcore_barrier()

    def prime(b, carry):
        pltpu.async_copy(y_hbm.at[src_v.at[b]], rbuf.at[b], gsem.at[b])
        return carry
    lax.fori_loop(0, NBUF, prime, 0)

    def step(g, carry):
        def ph1(b, carry):
            j = g * NBUF + b
            pltpu.make_async_copy(
                y_hbm.at[src_v.at[j]], rbuf.at[b], gsem.at[b]).wait()
            pltpu.async_copy(
                rbuf.at[b], acc_sh.at[dst_v.at[j]], ssem.at[b], add=True)
            return carry
        lax.fori_loop(0, NBUF, ph1, 0)

        def ph2(b, carry):
            j = g * NBUF + b
            pltpu.make_async_copy(
                rbuf.at[b], acc_sh.at[dst_v.at[j]], ssem.at[b]).wait()

            @pl.when(j + NBUF < WPW)
            def _issue_next():
                pltpu.async_copy(
                    y_hbm.at[src_v.at[j + NBUF]], rbuf.at[b], gsem.at[b])
            return carry
        lax.fori_loop(0, NBUF, ph2, 0)
        return carry
    lax.fori_loop(0, WPW // NBUF, step, 0)

    plsc.subcore_barrier()
    def wrd(q, carry):
        pltpu.async_copy(
            acc_sh.at[pl.ds(sid * RPT + q * K, K), :], rbuf.at[q],
            gsem.at[q])
        return carry
    lax.fori_loop(0, RPT // K, wrd, 0)

    def wwr(q, carry):
        pltpu.make_async_copy(
            acc_sh.at[pl.ds(sid * RPT + q * K, K), :], rbuf.at[q],
            gsem.at[q]).wait()
        pltpu.async_copy(
            rbuf.at[q], out_hbm.at[cid, pl.ds(sid * RPT + q * K, K), :],
            ssem.at[q])
        return carry
    lax.fori_loop(0, RPT // K, wwr, 0)

    def wdr(q, carry):
        pltpu.make_async_copy(
            rbuf.at[q], out_hbm.at[cid, pl.ds(sid * RPT + q * K, K), :],
            ssem.at[q]).wait()
        return carry
    lax.fori_loop(0, RPT // K, wdr, 0)


_conv_kernel = pl.kernel(
    _conv_body,
    out_type=jax.ShapeDtypeStruct((NC, TROWS, CW), F32),
    mesh=_mesh,
    compiler_params=pltpu.CompilerParams(use_tc_tiling_on_sc=False),
    scratch_types=[
        pltpu.VMEM((WPW, K), I32),
        pltpu.VMEM((WPW, K), I32),
        pltpu.VMEM_SHARED((TROWS, CW), F32),
        pltpu.VMEM((NBUF, K, CW), F32),
        pltpu.SemaphoreType.DMA((NBUF,)),
        pltpu.SemaphoreType.DMA((NBUF,)),
    ],
)


RB = 1000
GRID = N // RB


def _dinv_block(d0_ref, d1_ref):
    return lax.rsqrt(d0_ref[...] + d1_ref[...] + 1.0)


def _tc1_body(x_ref, d0_ref, d1_ref, w1_ref, *y_refs):
    dinv = _dinv_block(d0_ref, d1_ref)
    xw = jnp.dot(x_ref[...], w1_ref[...], preferred_element_type=F32)
    y = xw * dinv
    for c in range(4):
        y_refs[c][...] = y[:, c * CW:(c + 1) * CW]


def _tc2_body(s10, s11, s12, s13, y10, y11, y12, y13,
              d0_ref, d1_ref, b1_ref, w2_ref, y20_ref, y21_ref):
    dinv = _dinv_block(d0_ref, d1_ref)
    b1 = b1_ref[...]
    hs = []
    for c, (s, y) in enumerate(((s10, y10), (s11, y11), (s12, y12),
                                (s13, y13))):
        hs.append(jnp.maximum(
            dinv * (s[0] + s[1] + y[...]) + b1[:, c * CW:(c + 1) * CW], 0.0))
    h = jnp.concatenate(hs, axis=1)
    y2 = jnp.dot(h, w2_ref[...], preferred_element_type=F32) * dinv
    y20_ref[...] = y2[:, :CW]
    y21_ref[...] = y2[:, CW:]


def _tc3_body(s20, s21, y20, y21, d0_ref, d1_ref, b2_ref,
              f1w_ref, f1b_ref, f2w_ref, f2b_ref, o_ref):
    dinv = _dinv_block(d0_ref, d1_ref)
    zs = []
    for c, (s, y) in enumerate(((s20, y20), (s21, y21))):
        zs.append(jnp.maximum(
            dinv * (s[0] + s[1] + y[...])
            + b2_ref[..., c * CW:(c + 1) * CW], 0.0))
    z = jnp.concatenate(zs, axis=1)
    t = jnp.dot(z, f1w_ref[...], preferred_element_type=F32) + f1b_ref[...]
    p = jnp.where(t > 0.0, t, jnp.exp(t) - 1.0)
    o_ref[...] = jnp.dot(p, f2w_ref[...], preferred_element_type=F32) \
        + f2b_ref[...]


def _row_spec(shape):
    nd = len(shape)
    if nd == 1:
        return pl.BlockSpec((RB,), lambda i: (i,))
    if nd == 2:
        return pl.BlockSpec((RB, shape[1]), lambda i: (i, 0))
    return pl.BlockSpec((shape[0], RB, shape[2]), lambda i: (0, i, 0))


def _full_spec(shape):
    return pl.BlockSpec(shape, lambda i: (0,) * len(shape))


def _tc_call(body, ins_row, ins_full, out_shapes):
    in_specs = [_row_spec(a.shape) for a in ins_row] + \
               [_full_spec(a.shape) for a in ins_full]
    out_specs = [_row_spec(s.shape) for s in out_shapes]
    outs = pl.pallas_call(
        body,
        grid=(GRID,),
        in_specs=in_specs,
        out_specs=out_specs if len(out_shapes) > 1 else out_specs[0],
        out_shape=out_shapes if len(out_shapes) > 1 else out_shapes[0],
    )(*ins_row, *ins_full)
    return outs


def kernel(x, edge_index, W1, b1, W2, b2, fc1_W, fc1_b, fc2_W, fc2_b):
    ei = edge_index.astype(I32)
    src, dst = ei[0], ei[1]
    pad = EP - E
    fk = jnp.arange(pad, dtype=I32)
    fake_src = (fk * 131) % N
    fake_dst = N + (fk % 128)
    srcp = jnp.concatenate([src, fake_src]).reshape(ROWS, K)
    dstp = jnp.concatenate([dst, fake_dst]).reshape(ROWS, K)

    d0, d1 = _deg_kernel(dstp)
    d0 = d0.reshape(TROWS, 1)
    d1 = d1.reshape(TROWS, 1)

    sds = jax.ShapeDtypeStruct
    y1 = _tc_call(
        _tc1_body, [x, d0, d1], [W1],
        [sds((N, CW), F32)] * 4)

    s1 = [_conv_kernel(srcp, dstp, yc) for yc in y1]

    y2 = _tc_call(
        _tc2_body, list(s1) + list(y1) + [d0, d1],
        [b1.reshape(1, D_HID), W2],
        [sds((N, CW), F32)] * 2)

    s2 = [_conv_kernel(srcp, dstp, yc) for yc in y2]

    out = _tc_call(
        _tc3_body, list(s2) + list(y2) + [d0, d1],
        [b2.reshape(1, 128), fc1_W, fc1_b.reshape(1, 128),
         fc2_W, fc2_b.reshape(1, 128)],
        [sds((N, D_OUT), F32)])
    return out


# speedup vs baseline: 25.1083x; 1.0574x over previous
import functools

import jax
import jax.numpy as jnp
from jax import lax
from jax.experimental import pallas as pl
from jax.experimental.pallas import tpu as pltpu
from jax.experimental.pallas import tpu_sc as plsc

F32 = jnp.float32
I32 = jnp.int32

N = 10000
E = 320000
D_HID = 256
D_OUT = 128
CW = 64

NC, NS = 2, 16
NW = NC * NS
K = 128
WPW = 80
CWPW = 160
EP = NW * WPW * K
ROWS = EP // K
TROWS = 10240
RPT = TROWS // NS
NBUF = 5
DBUF = 8

_mesh = plsc.VectorSubcoreMesh(core_axis_name="c", subcore_axis_name="s")


def _deg_body(dst_hbm, deg0_out, deg1_out, deg_sh, dst_v, ones_v, wb_v,
              dsem):
    cid = lax.axis_index("c")
    sid = lax.axis_index("s")
    wid = cid * NS + sid

    for l in range(8):
        ones_v[pl.ds(l * 16, 16)] = jnp.full((16,), 1.0, F32)

    def zr(i, carry):
        wb_v[pl.ds(i * 16, 16)] = jnp.zeros((16,), F32)
        return carry
    lax.fori_loop(0, RPT // 16, zr, 0)

    pltpu.sync_copy(wb_v, deg_sh.at[pl.ds(sid * RPT, RPT)])
    plsc.subcore_barrier()

    pltpu.sync_copy(dst_hbm.at[pl.ds(wid * WPW, WPW)], dst_v)

    def step(g, carry):
        for b in range(DBUF):
            j = g * DBUF + b
            pltpu.async_copy(ones_v, deg_sh.at[dst_v.at[j]], dsem, add=True)
        for b in range(DBUF):
            j = g * DBUF + b
            pltpu.make_async_copy(ones_v, deg_sh.at[dst_v.at[j]],
                                  dsem).wait()
        return carry
    lax.fori_loop(0, WPW // DBUF, step, 0)

    plsc.subcore_barrier()
    pltpu.sync_copy(deg_sh.at[pl.ds(sid * RPT, RPT)], wb_v)

    @pl.when(cid == 0)
    def _wb0():
        pltpu.sync_copy(wb_v, deg0_out.at[pl.ds(sid * RPT, RPT)])

    @pl.when(cid == 1)
    def _wb1():
        pltpu.sync_copy(wb_v, deg1_out.at[pl.ds(sid * RPT, RPT)])


_deg_kernel = pl.kernel(
    _deg_body,
    out_type=[jax.ShapeDtypeStruct((TROWS,), F32)] * 2,
    mesh=_mesh,
    scratch_types=[
        pltpu.VMEM_SHARED((TROWS,), F32),
        pltpu.VMEM((WPW, K), I32),
        pltpu.VMEM((K,), F32),
        pltpu.VMEM((RPT,), F32),
        pltpu.SemaphoreType.DMA,
    ],
)


def _conv_body(src_hbm, dst_hbm, y_hbm, out_hbm, src_v, dst_v, acc_sh,
               rbuf, gsem, ssem):
    cid = lax.axis_index("c")
    sid = lax.axis_index("s")

    cp_s = pltpu.async_copy(src_hbm.at[pl.ds(sid * CWPW, CWPW)], src_v,
                            gsem.at[0])
    cp_d = pltpu.async_copy(dst_hbm.at[pl.ds(sid * CWPW, CWPW)], dst_v,
                            gsem.at[1])

    def zrow(i, carry):
        for l in range(CW // 16):
            rbuf[0, i, pl.ds(l * 16, 16)] = jnp.zeros((16,), F32)
        return carry
    lax.fori_loop(0, K, zrow, 0)

    def zcp(q, carry):
        pltpu.async_copy(rbuf.at[0],
                         acc_sh.at[pl.ds(sid * RPT + q * K, K), :],
                         ssem.at[0])
        return carry
    lax.fori_loop(0, RPT // K, zcp, 0)

    def zdr(q, carry):
        pltpu.make_async_copy(
            rbuf.at[0], acc_sh.at[pl.ds(sid * RPT, K), :], ssem.at[0]).wait()
        return carry
    lax.fori_loop(0, RPT // K, zdr, 0)
    cp_s.wait()
    cp_d.wait()
    plsc.subcore_barrier()

    yv = y_hbm.at[cid]
    ov = out_hbm.at[cid]

    def prime(b, carry):
        pltpu.async_copy(yv.at[src_v.at[b]], rbuf.at[b], gsem.at[b])
        return carry
    lax.fori_loop(0, NBUF, prime, 0)

    def step(g, carry):
        def ph1(b, carry):
            j = g * NBUF + b
            pltpu.make_async_copy(
                yv.at[src_v.at[j]], rbuf.at[b], gsem.at[b]).wait()
            pltpu.async_copy(
                rbuf.at[b], acc_sh.at[dst_v.at[j]], ssem.at[b], add=True)
            return carry
        lax.fori_loop(0, NBUF, ph1, 0)

        def ph2(b, carry):
            j = g * NBUF + b
            pltpu.make_async_copy(
                rbuf.at[b], acc_sh.at[dst_v.at[j]], ssem.at[b]).wait()

            @pl.when(j + NBUF < CWPW)
            def _issue_next():
                pltpu.async_copy(
                    yv.at[src_v.at[j + NBUF]], rbuf.at[b], gsem.at[b])
            return carry
        lax.fori_loop(0, NBUF, ph2, 0)
        return carry
    lax.fori_loop(0, CWPW // NBUF, step, 0)

    plsc.subcore_barrier()
    def wrd(q, carry):
        pltpu.async_copy(
            acc_sh.at[pl.ds(sid * RPT + q * K, K), :], rbuf.at[q],
            gsem.at[q])
        return carry
    lax.fori_loop(0, RPT // K, wrd, 0)

    def wwr(q, carry):
        pltpu.make_async_copy(
            acc_sh.at[pl.ds(sid * RPT + q * K, K), :], rbuf.at[q],
            gsem.at[q]).wait()
        pltpu.async_copy(
            rbuf.at[q], ov.at[pl.ds(sid * RPT + q * K, K), :],
            ssem.at[q])
        return carry
    lax.fori_loop(0, RPT // K, wwr, 0)

    def wdr(q, carry):
        pltpu.make_async_copy(
            rbuf.at[q], ov.at[pl.ds(sid * RPT + q * K, K), :],
            ssem.at[q]).wait()
        return carry
    lax.fori_loop(0, RPT // K, wdr, 0)


_conv_kernel = pl.kernel(
    _conv_body,
    out_type=jax.ShapeDtypeStruct((NC, TROWS, CW), F32),
    mesh=_mesh,
    compiler_params=pltpu.CompilerParams(use_tc_tiling_on_sc=False),
    scratch_types=[
        pltpu.VMEM((CWPW, K), I32),
        pltpu.VMEM((CWPW, K), I32),
        pltpu.VMEM_SHARED((TROWS, CW), F32),
        pltpu.VMEM((NBUF, K, CW), F32),
        pltpu.SemaphoreType.DMA((NBUF,)),
        pltpu.SemaphoreType.DMA((NBUF,)),
    ],
)


RB = 1000
GRID = N // RB


def _dinv_block(d0_ref, d1_ref):
    return lax.rsqrt(d0_ref[...] + d1_ref[...] + 1.0)


def _tc1_body(x_ref, d0_ref, d1_ref, w1_ref, ya_ref, yb_ref):
    dinv = _dinv_block(d0_ref, d1_ref)
    xw = jnp.dot(x_ref[...], w1_ref[...], preferred_element_type=F32)
    y = xw * dinv
    ya_ref[0] = y[:, 0 * CW:1 * CW]
    ya_ref[1] = y[:, 1 * CW:2 * CW]
    yb_ref[0] = y[:, 2 * CW:3 * CW]
    yb_ref[1] = y[:, 3 * CW:4 * CW]


def _tc2_body(s1a, s1b, y1a, y1b, d0_ref, d1_ref, b1_ref, w2_ref,
              y2_ref):
    dinv = _dinv_block(d0_ref, d1_ref)
    b1 = b1_ref[...]
    hs = []
    for c in range(4):
        s = (s1a, s1b)[c // 2][c % 2]
        y = (y1a, y1b)[c // 2][c % 2]
        hs.append(jnp.maximum(
            dinv * (s + y) + b1[:, c * CW:(c + 1) * CW], 0.0))
    h = jnp.concatenate(hs, axis=1)
    y2 = jnp.dot(h, w2_ref[...], preferred_element_type=F32) * dinv
    y2_ref[0] = y2[:, :CW]
    y2_ref[1] = y2[:, CW:]


def _tc3_body(s2, y2, d0_ref, d1_ref, b2_ref,
              f1w_ref, f1b_ref, f2w_ref, f2b_ref, o_ref):
    dinv = _dinv_block(d0_ref, d1_ref)
    zs = []
    for c in range(2):
        zs.append(jnp.maximum(
            dinv * (s2[c] + y2[c])
            + b2_ref[..., c * CW:(c + 1) * CW], 0.0))
    z = jnp.concatenate(zs, axis=1)
    t = jnp.dot(z, f1w_ref[...], preferred_element_type=F32) + f1b_ref[...]
    p = jnp.where(t > 0.0, t, jnp.exp(t) - 1.0)
    o_ref[...] = jnp.dot(p, f2w_ref[...], preferred_element_type=F32) \
        + f2b_ref[...]


def _row_spec(shape):
    nd = len(shape)
    if nd == 1:
        return pl.BlockSpec((RB,), lambda i: (i,))
    if nd == 2:
        return pl.BlockSpec((RB, shape[1]), lambda i: (i, 0))
    return pl.BlockSpec((shape[0], RB, shape[2]), lambda i: (0, i, 0))


def _full_spec(shape):
    return pl.BlockSpec(shape, lambda i: (0,) * len(shape))


def _tc_call(body, ins_row, ins_full, out_shapes):
    in_specs = [_row_spec(a.shape) for a in ins_row] + \
               [_full_spec(a.shape) for a in ins_full]
    out_specs = [_row_spec(s.shape) for s in out_shapes]
    outs = pl.pallas_call(
        body,
        grid=(GRID,),
        in_specs=in_specs,
        out_specs=out_specs if len(out_shapes) > 1 else out_specs[0],
        out_shape=out_shapes if len(out_shapes) > 1 else out_shapes[0],
    )(*ins_row, *ins_full)
    return outs


def kernel(x, edge_index, W1, b1, W2, b2, fc1_W, fc1_b, fc2_W, fc2_b):
    ei = edge_index.astype(I32)
    src, dst = ei[0], ei[1]
    pad = EP - E
    fk = jnp.arange(pad, dtype=I32)
    fake_src = (fk * 131) % N
    fake_dst = N + (fk % 128)
    srcp = jnp.concatenate([src, fake_src]).reshape(ROWS, K)
    dstp = jnp.concatenate([dst, fake_dst]).reshape(ROWS, K)

    d0, d1 = _deg_kernel(dstp)
    d0 = d0.reshape(TROWS, 1)
    d1 = d1.reshape(TROWS, 1)

    sds = jax.ShapeDtypeStruct
    y1a, y1b = _tc_call(
        _tc1_body, [x, d0, d1], [W1],
        [sds((2, N, CW), F32)] * 2)

    s1a = _conv_kernel(srcp, dstp, y1a)
    s1b = _conv_kernel(srcp, dstp, y1b)

    y2 = _tc_call(
        _tc2_body, [s1a, s1b, y1a, y1b, d0, d1],
        [b1.reshape(1, D_HID), W2],
        [sds((2, N, CW), F32)])

    s2 = _conv_kernel(srcp, dstp, y2)

    out = _tc_call(
        _tc3_body, [s2, y2, d0, d1],
        [b2.reshape(1, 128), fc1_W, fc1_b.reshape(1, 128),
         fc2_W, fc2_b.reshape(1, 128)],
        [sds((N, D_OUT), F32)])
    return out


# speedup vs baseline: 26.4691x; 1.0542x over previous
import functools

import jax
import jax.numpy as jnp
from jax import lax
from jax.experimental import pallas as pl
from jax.experimental.pallas import tpu as pltpu
from jax.experimental.pallas import tpu_sc as plsc

F32 = jnp.float32
I32 = jnp.int32

N = 10000
E = 320000
D_HID = 256
D_OUT = 128

NC, NS = 2, 16
NW = NC * NS
K = 128
WPW = 81
CWPW = 162
EP = NW * WPW * K
ROWS = EP // K
TROWS = 10032
RPT = TROWS // NS
NWB = 11
WBR = RPT // NWB
DTROWS = 10240
DRPT = DTROWS // NS
NBUF = 3
DBUF = 9

_mesh = plsc.VectorSubcoreMesh(core_axis_name="c", subcore_axis_name="s")


def _deg_body(dst_hbm, deg0_out, deg1_out, deg_sh, dst_v, ones_v, wb_v,
              dsem):
    cid = lax.axis_index("c")
    sid = lax.axis_index("s")
    wid = cid * NS + sid

    for l in range(8):
        ones_v[pl.ds(l * 16, 16)] = jnp.full((16,), 1.0, F32)

    def zr(i, carry):
        wb_v[pl.ds(i * 16, 16)] = jnp.zeros((16,), F32)
        return carry
    lax.fori_loop(0, DRPT // 16, zr, 0)

    pltpu.sync_copy(wb_v, deg_sh.at[pl.ds(sid * DRPT, DRPT)])
    plsc.subcore_barrier()

    pltpu.sync_copy(dst_hbm.at[pl.ds(wid * WPW, WPW)], dst_v)

    def step(g, carry):
        for b in range(DBUF):
            j = g * DBUF + b
            pltpu.async_copy(ones_v, deg_sh.at[dst_v.at[j]], dsem, add=True)
        for b in range(DBUF):
            j = g * DBUF + b
            pltpu.make_async_copy(ones_v, deg_sh.at[dst_v.at[j]],
                                  dsem).wait()
        return carry
    lax.fori_loop(0, WPW // DBUF, step, 0)

    plsc.subcore_barrier()
    pltpu.sync_copy(deg_sh.at[pl.ds(sid * DRPT, DRPT)], wb_v)

    @pl.when(cid == 0)
    def _wb0():
        pltpu.sync_copy(wb_v, deg0_out.at[pl.ds(sid * DRPT, DRPT)])

    @pl.when(cid == 1)
    def _wb1():
        pltpu.sync_copy(wb_v, deg1_out.at[pl.ds(sid * DRPT, DRPT)])


_deg_kernel = pl.kernel(
    _deg_body,
    out_type=[jax.ShapeDtypeStruct((DTROWS,), F32)] * 2,
    mesh=_mesh,
    compiler_params=pltpu.CompilerParams(use_tc_tiling_on_sc=False),
    scratch_types=[
        pltpu.VMEM_SHARED((DTROWS,), F32),
        pltpu.VMEM((WPW, K), I32),
        pltpu.VMEM((K,), F32),
        pltpu.VMEM((DRPT,), F32),
        pltpu.SemaphoreType.DMA,
    ],
)


def _conv_body(split_edges, src_hbm, dst_hbm, y_hbm, out_hbm,
               src_v, dst_v, acc_sh, rbuf, gsem, ssem, isem):
    cid = lax.axis_index("c")
    sid = lax.axis_index("s")
    if split_edges:
        nwin = WPW
        slab = (cid * NS + sid) * WPW
        yv = y_hbm
    else:
        nwin = CWPW
        slab = sid * CWPW
        yv = y_hbm.at[cid]
    ngrp = nwin // NBUF
    ov = out_hbm.at[cid]

    pltpu.async_copy(src_hbm.at[pl.ds(slab, NBUF)], src_v.at[0], isem.at[0])
    pltpu.async_copy(dst_hbm.at[pl.ds(slab, NBUF)], dst_v.at[0], isem.at[0])

    def zrow(i, carry):
        for l in range(8):
            rbuf[0, i, pl.ds(l * 16, 16)] = jnp.zeros((16,), F32)
        return carry
    lax.fori_loop(0, WBR, zrow, 0)

    def zcp(q, carry):
        pltpu.async_copy(rbuf.at[0, pl.ds(0, WBR)],
                         acc_sh.at[pl.ds(sid * RPT + q * WBR, WBR), :],
                         ssem.at[0])
        return carry
    lax.fori_loop(0, NWB, zcp, 0)

    def zdr(q, carry):
        pltpu.make_async_copy(
            rbuf.at[0, pl.ds(0, WBR)],
            acc_sh.at[pl.ds(sid * RPT, WBR), :], ssem.at[0]).wait()
        return carry
    lax.fori_loop(0, NWB, zdr, 0)
    plsc.subcore_barrier()

    pltpu.make_async_copy(src_hbm.at[pl.ds(slab, NBUF)], src_v.at[0],
                          isem.at[0]).wait()
    pltpu.make_async_copy(dst_hbm.at[pl.ds(slab, NBUF)], dst_v.at[0],
                          isem.at[0]).wait()
    pltpu.async_copy(src_hbm.at[pl.ds(slab + NBUF, NBUF)], src_v.at[1],
                     isem.at[1])
    pltpu.async_copy(dst_hbm.at[pl.ds(slab + NBUF, NBUF)], dst_v.at[1],
                     isem.at[1])

    def prime(b, carry):
        pltpu.async_copy(yv.at[src_v.at[0, b]], rbuf.at[b], gsem.at[b])
        return carry
    lax.fori_loop(0, NBUF, prime, 0)

    def step(g, carry):
        gb = lax.rem(g, 2)
        nb = lax.rem(g + 1, 2)

        def ph1(b, carry):
            pltpu.make_async_copy(
                yv.at[src_v.at[gb, b]], rbuf.at[b], gsem.at[b]).wait()
            pltpu.async_copy(
                rbuf.at[b], acc_sh.at[dst_v.at[gb, b]], ssem.at[b],
                add=True)
            return carry
        lax.fori_loop(0, NBUF, ph1, 0)

        @pl.when(g + 1 < ngrp)
        def _wi():
            pltpu.make_async_copy(
                src_hbm.at[pl.ds(slab, NBUF)], src_v.at[nb],
                isem.at[nb]).wait()
            pltpu.make_async_copy(
                dst_hbm.at[pl.ds(slab, NBUF)], dst_v.at[nb],
                isem.at[nb]).wait()

        def ph2(b, carry):
            pltpu.make_async_copy(
                rbuf.at[b], acc_sh.at[dst_v.at[gb, b]], ssem.at[b]).wait()

            @pl.when(g + 1 < ngrp)
            def _issue_next():
                pltpu.async_copy(
                    yv.at[src_v.at[nb, b]], rbuf.at[b], gsem.at[b])
            return carry
        lax.fori_loop(0, NBUF, ph2, 0)

        @pl.when(g + 2 < ngrp)
        def _pf():
            base = slab + (g + 2) * NBUF
            pltpu.async_copy(src_hbm.at[pl.ds(base, NBUF)], src_v.at[gb],
                             isem.at[gb])
            pltpu.async_copy(dst_hbm.at[pl.ds(base, NBUF)], dst_v.at[gb],
                             isem.at[gb])
        return carry
    lax.fori_loop(0, ngrp, step, 0)

    plsc.subcore_barrier()
    def wprime(q, carry):
        pltpu.async_copy(
            acc_sh.at[pl.ds(sid * RPT + q * WBR, WBR), :],
            rbuf.at[lax.rem(q, NBUF), pl.ds(0, WBR)],
            gsem.at[lax.rem(q, NBUF)])
        return carry
    lax.fori_loop(0, NBUF, wprime, 0)

    def wchain(q, carry):
        b = lax.rem(q, NBUF)
        pltpu.make_async_copy(
            acc_sh.at[pl.ds(sid * RPT, WBR), :],
            rbuf.at[b, pl.ds(0, WBR)], gsem.at[b]).wait()
        pltpu.async_copy(
            rbuf.at[b, pl.ds(0, WBR)],
            ov.at[pl.ds(sid * RPT + q * WBR, WBR), :], ssem.at[b])

        @pl.when(q + NBUF < NWB)
        def _next_rd():
            pltpu.make_async_copy(
                rbuf.at[b, pl.ds(0, WBR)],
                ov.at[pl.ds(sid * RPT, WBR), :], ssem.at[b]).wait()
            pltpu.async_copy(
                acc_sh.at[pl.ds(sid * RPT + (q + NBUF) * WBR, WBR), :],
                rbuf.at[b, pl.ds(0, WBR)], gsem.at[b])
        return carry
    lax.fori_loop(0, NWB, wchain, 0)

    def wdrain(q, carry):
        b = lax.rem(q, NBUF)
        pltpu.make_async_copy(
            rbuf.at[b, pl.ds(0, WBR)],
            ov.at[pl.ds(sid * RPT, WBR), :], ssem.at[b]).wait()
        return carry
    lax.fori_loop(NWB - NBUF, NWB, wdrain, 0)


def _make_conv(split_edges, y_shape):
    return pl.kernel(
        functools.partial(_conv_body, split_edges),
        out_type=jax.ShapeDtypeStruct((NC, TROWS, 128), F32),
        mesh=_mesh,
        compiler_params=pltpu.CompilerParams(use_tc_tiling_on_sc=False),
        scratch_types=[
            pltpu.VMEM((2, NBUF, K), I32),
            pltpu.VMEM((2, NBUF, K), I32),
            pltpu.VMEM_SHARED((TROWS, 128), F32),
            pltpu.VMEM((NBUF, K, 128), F32),
            pltpu.SemaphoreType.DMA((NBUF,)),
            pltpu.SemaphoreType.DMA((NBUF,)),
            pltpu.SemaphoreType.DMA((2,)),
        ],
    )


_conv1_kernel = _make_conv(False, (NC, N, 128))
_conv2_kernel = _make_conv(True, (N, 128))


RB = 1000
GRID = N // RB


def _dinv_block(d0_ref, d1_ref):
    return lax.rsqrt(d0_ref[...] + d1_ref[...] + 1.0)


def _tc1_body(x_ref, d0_ref, d1_ref, w1_ref, y1_ref):
    dinv = _dinv_block(d0_ref, d1_ref)
    xw = jnp.dot(x_ref[...], w1_ref[...], preferred_element_type=F32)
    y = xw * dinv
    y1_ref[0] = y[:, :128]
    y1_ref[1] = y[:, 128:]


def _tc2_body(s1_ref, y1_ref, d0_ref, d1_ref, b1_ref, w2_ref, y2_ref):
    dinv = _dinv_block(d0_ref, d1_ref)
    b1 = b1_ref[...]
    h0 = jnp.maximum(dinv * (s1_ref[0] + y1_ref[0]) + b1[:, :128], 0.0)
    h1 = jnp.maximum(dinv * (s1_ref[1] + y1_ref[1]) + b1[:, 128:], 0.0)
    h = jnp.concatenate([h0, h1], axis=1)
    y2_ref[...] = jnp.dot(h, w2_ref[...], preferred_element_type=F32) * dinv


def _tc3_body(s2_ref, y2_ref, d0_ref, d1_ref, b2_ref,
              f1w_ref, f1b_ref, f2w_ref, f2b_ref, o_ref):
    dinv = _dinv_block(d0_ref, d1_ref)
    z = jnp.maximum(dinv * (s2_ref[0] + s2_ref[1] + y2_ref[...])
                    + b2_ref[...], 0.0)
    t = jnp.dot(z, f1w_ref[...], preferred_element_type=F32) + f1b_ref[...]
    p = jnp.where(t > 0.0, t, jnp.exp(t) - 1.0)
    o_ref[...] = jnp.dot(p, f2w_ref[...], preferred_element_type=F32) \
        + f2b_ref[...]


def _row_spec(shape):
    nd = len(shape)
    if nd == 2:
        return pl.BlockSpec((RB, shape[1]), lambda i: (i, 0))
    return pl.BlockSpec((shape[0], RB, shape[2]), lambda i: (0, i, 0))


def _full_spec(shape):
    return pl.BlockSpec(shape, lambda i: (0,) * len(shape))


def _tc_call(body, ins_row, ins_full, out_shapes):
    in_specs = [_row_spec(a.shape) for a in ins_row] + \
               [_full_spec(a.shape) for a in ins_full]
    out_specs = [_row_spec(s.shape) for s in out_shapes]
    outs = pl.pallas_call(
        body,
        grid=(GRID,),
        in_specs=in_specs,
        out_specs=out_specs if len(out_shapes) > 1 else out_specs[0],
        out_shape=out_shapes if len(out_shapes) > 1 else out_shapes[0],
    )(*ins_row, *ins_full)
    return outs


def kernel(x, edge_index, W1, b1, W2, b2, fc1_W, fc1_b, fc2_W, fc2_b):
    ei = edge_index.astype(I32)
    src, dst = ei[0], ei[1]
    pad = EP - E
    fk = jnp.arange(pad, dtype=I32)
    fake_src = (fk * 131) % N
    fake_dst = N + (fk % 32)
    srcp = jnp.concatenate([src, fake_src]).reshape(ROWS, K)
    dstp = jnp.concatenate([dst, fake_dst]).reshape(ROWS, K)

    d0, d1 = _deg_kernel(dstp)
    d0 = d0.reshape(DTROWS, 1)
    d1 = d1.reshape(DTROWS, 1)

    sds = jax.ShapeDtypeStruct
    y1 = _tc_call(
        _tc1_body, [x, d0, d1], [W1],
        [sds((NC, N, 128), F32)])

    s1 = _conv1_kernel(srcp, dstp, y1)

    y2 = _tc_call(
        _tc2_body, [s1, y1, d0, d1],
        [b1.reshape(1, D_HID), W2],
        [sds((N, 128), F32)])

    s2 = _conv2_kernel(srcp, dstp, y2)

    out = _tc_call(
        _tc3_body, [s2, y2, d0, d1],
        [b2.reshape(1, 128), fc1_W, fc1_b.reshape(1, 128),
         fc2_W, fc2_b.reshape(1, 128)],
        [sds((N, D_OUT), F32)])
    return out


# speedup vs baseline: 28.8219x; 1.0889x over previous
import functools

import jax
import jax.numpy as jnp
from jax import lax
from jax.experimental import pallas as pl
from jax.experimental.pallas import tpu as pltpu
from jax.experimental.pallas import tpu_sc as plsc

F32 = jnp.float32
I32 = jnp.int32

N = 10000
E = 320000
D_HID = 256
D_OUT = 128

NC, NS = 2, 16
NW = NC * NS
K = 64
WPW = 162
CWPW = 324
EP = NW * WPW * K
ROWS = EP // K
TROWS = 10032
RPT = TROWS // NS
NWB = 11
WBR = RPT // NWB
DTROWS = 10240
DRPT = DTROWS // NS
NBUF = 6
DBUF = 9

_mesh = plsc.VectorSubcoreMesh(core_axis_name="c", subcore_axis_name="s")


def _deg_body(dst_hbm, deg0_out, deg1_out, deg_sh, dst_v, ones_v, wb_v,
              dsem):
    cid = lax.axis_index("c")
    sid = lax.axis_index("s")
    wid = cid * NS + sid

    for l in range(K // 16):
        ones_v[pl.ds(l * 16, 16)] = jnp.full((16,), 1.0, F32)

    def zr(i, carry):
        wb_v[pl.ds(i * 16, 16)] = jnp.zeros((16,), F32)
        return carry
    lax.fori_loop(0, DRPT // 16, zr, 0)

    pltpu.sync_copy(wb_v, deg_sh.at[pl.ds(sid * DRPT, DRPT)])
    plsc.subcore_barrier()

    pltpu.sync_copy(dst_hbm.at[pl.ds(wid * WPW, WPW)], dst_v)

    def step(g, carry):
        for b in range(DBUF):
            j = g * DBUF + b
            pltpu.async_copy(ones_v, deg_sh.at[dst_v.at[j]], dsem, add=True)
        for b in range(DBUF):
            j = g * DBUF + b
            pltpu.make_async_copy(ones_v, deg_sh.at[dst_v.at[j]],
                                  dsem).wait()
        return carry
    lax.fori_loop(0, WPW // DBUF, step, 0)

    plsc.subcore_barrier()
    pltpu.sync_copy(deg_sh.at[pl.ds(sid * DRPT, DRPT)], wb_v)

    @pl.when(cid == 0)
    def _wb0():
        pltpu.sync_copy(wb_v, deg0_out.at[pl.ds(sid * DRPT, DRPT)])

    @pl.when(cid == 1)
    def _wb1():
        pltpu.sync_copy(wb_v, deg1_out.at[pl.ds(sid * DRPT, DRPT)])


_deg_kernel = pl.kernel(
    _deg_body,
    out_type=[jax.ShapeDtypeStruct((DTROWS,), F32)] * 2,
    mesh=_mesh,
    compiler_params=pltpu.CompilerParams(use_tc_tiling_on_sc=False),
    scratch_types=[
        pltpu.VMEM_SHARED((DTROWS,), F32),
        pltpu.VMEM((WPW, K), I32),
        pltpu.VMEM((K,), F32),
        pltpu.VMEM((DRPT,), F32),
        pltpu.SemaphoreType.DMA,
    ],
)


def _conv_body(split_edges, src_hbm, dst_hbm, y_hbm, out_hbm,
               src_v, dst_v, acc_sh, rbuf, gsem, ssem, isem):
    cid = lax.axis_index("c")
    sid = lax.axis_index("s")
    if split_edges:
        nwin = WPW
        slab = (cid * NS + sid) * WPW
        yv = y_hbm
    else:
        nwin = CWPW
        slab = sid * CWPW
        yv = y_hbm.at[cid]
    ngrp = nwin // NBUF
    ov = out_hbm.at[cid]

    pltpu.async_copy(src_hbm.at[pl.ds(slab, NBUF)], src_v.at[0], isem.at[0])
    pltpu.async_copy(dst_hbm.at[pl.ds(slab, NBUF)], dst_v.at[0], isem.at[0])

    def zrow(i, carry):
        for l in range(8):
            rbuf[0, i, pl.ds(l * 16, 16)] = jnp.zeros((16,), F32)
        return carry
    lax.fori_loop(0, WBR, zrow, 0)

    def zcp(q, carry):
        pltpu.async_copy(rbuf.at[0, pl.ds(0, WBR)],
                         acc_sh.at[pl.ds(sid * RPT + q * WBR, WBR), :],
                         ssem.at[0])
        return carry
    lax.fori_loop(0, NWB, zcp, 0)

    def zdr(q, carry):
        pltpu.make_async_copy(
            rbuf.at[0, pl.ds(0, WBR)],
            acc_sh.at[pl.ds(sid * RPT, WBR), :], ssem.at[0]).wait()
        return carry
    lax.fori_loop(0, NWB, zdr, 0)
    plsc.subcore_barrier()

    pltpu.make_async_copy(src_hbm.at[pl.ds(slab, NBUF)], src_v.at[0],
                          isem.at[0]).wait()
    pltpu.make_async_copy(dst_hbm.at[pl.ds(slab, NBUF)], dst_v.at[0],
                          isem.at[0]).wait()
    pltpu.async_copy(src_hbm.at[pl.ds(slab + NBUF, NBUF)], src_v.at[1],
                     isem.at[1])
    pltpu.async_copy(dst_hbm.at[pl.ds(slab + NBUF, NBUF)], dst_v.at[1],
                     isem.at[1])

    def prime(b, carry):
        pltpu.async_copy(yv.at[src_v.at[0, b]], rbuf.at[b], gsem.at[b])
        return carry
    lax.fori_loop(0, NBUF, prime, 0)

    def step(g, carry):
        gb = lax.rem(g, 2)
        nb = lax.rem(g + 1, 2)

        def ph1(b, carry):
            pltpu.make_async_copy(
                yv.at[src_v.at[gb, b]], rbuf.at[b], gsem.at[b]).wait()
            pltpu.async_copy(
                rbuf.at[b], acc_sh.at[dst_v.at[gb, b]], ssem.at[b],
                add=True)
            return carry
        lax.fori_loop(0, NBUF, ph1, 0)

        @pl.when(g + 1 < ngrp)
        def _wi():
            pltpu.make_async_copy(
                src_hbm.at[pl.ds(slab, NBUF)], src_v.at[nb],
                isem.at[nb]).wait()
            pltpu.make_async_copy(
                dst_hbm.at[pl.ds(slab, NBUF)], dst_v.at[nb],
                isem.at[nb]).wait()

        def ph2(b, carry):
            pltpu.make_async_copy(
                rbuf.at[b], acc_sh.at[dst_v.at[gb, b]], ssem.at[b]).wait()

            @pl.when(g + 1 < ngrp)
            def _issue_next():
                pltpu.async_copy(
                    yv.at[src_v.at[nb, b]], rbuf.at[b], gsem.at[b])
            return carry
        lax.fori_loop(0, NBUF, ph2, 0)

        @pl.when(g + 2 < ngrp)
        def _pf():
            base = slab + (g + 2) * NBUF
            pltpu.async_copy(src_hbm.at[pl.ds(base, NBUF)], src_v.at[gb],
                             isem.at[gb])
            pltpu.async_copy(dst_hbm.at[pl.ds(base, NBUF)], dst_v.at[gb],
                             isem.at[gb])
        return carry
    lax.fori_loop(0, ngrp, step, 0)

    plsc.subcore_barrier()
    def wprime(q, carry):
        pltpu.async_copy(
            acc_sh.at[pl.ds(sid * RPT + q * WBR, WBR), :],
            rbuf.at[lax.rem(q, NBUF), pl.ds(0, WBR)],
            gsem.at[lax.rem(q, NBUF)])
        return carry
    lax.fori_loop(0, NBUF, wprime, 0)

    def wchain(q, carry):
        b = lax.rem(q, NBUF)
        pltpu.make_async_copy(
            acc_sh.at[pl.ds(sid * RPT, WBR), :],
            rbuf.at[b, pl.ds(0, WBR)], gsem.at[b]).wait()
        pltpu.async_copy(
            rbuf.at[b, pl.ds(0, WBR)],
            ov.at[pl.ds(sid * RPT + q * WBR, WBR), :], ssem.at[b])

        @pl.when(q + NBUF < NWB)
        def _next_rd():
            pltpu.make_async_copy(
                rbuf.at[b, pl.ds(0, WBR)],
                ov.at[pl.ds(sid * RPT, WBR), :], ssem.at[b]).wait()
            pltpu.async_copy(
                acc_sh.at[pl.ds(sid * RPT + (q + NBUF) * WBR, WBR), :],
                rbuf.at[b, pl.ds(0, WBR)], gsem.at[b])
        return carry
    lax.fori_loop(0, NWB, wchain, 0)

    def wdrain(q, carry):
        b = lax.rem(q, NBUF)
        pltpu.make_async_copy(
            rbuf.at[b, pl.ds(0, WBR)],
            ov.at[pl.ds(sid * RPT, WBR), :], ssem.at[b]).wait()
        return carry
    lax.fori_loop(NWB - NBUF, NWB, wdrain, 0)


def _make_conv(split_edges, y_shape):
    return pl.kernel(
        functools.partial(_conv_body, split_edges),
        out_type=jax.ShapeDtypeStruct((NC, TROWS, 128), F32),
        mesh=_mesh,
        compiler_params=pltpu.CompilerParams(use_tc_tiling_on_sc=False),
        scratch_types=[
            pltpu.VMEM((2, NBUF, K), I32),
            pltpu.VMEM((2, NBUF, K), I32),
            pltpu.VMEM_SHARED((TROWS, 128), F32),
            pltpu.VMEM((NBUF, K, 128), F32),
            pltpu.SemaphoreType.DMA((NBUF,)),
            pltpu.SemaphoreType.DMA((NBUF,)),
            pltpu.SemaphoreType.DMA((2,)),
        ],
    )


_conv1_kernel = _make_conv(False, (NC, N, 128))
_conv2_kernel = _make_conv(True, (N, 128))


RB = 1000
GRID = N // RB


def _dinv_block(d0_ref, d1_ref):
    return lax.rsqrt(d0_ref[...] + d1_ref[...] + 1.0)


def _tc1_body(x_ref, d0_ref, d1_ref, w1_ref, y1_ref):
    dinv = _dinv_block(d0_ref, d1_ref)
    xw = jnp.dot(x_ref[...], w1_ref[...], preferred_element_type=F32)
    y = xw * dinv
    y1_ref[0] = y[:, :128]
    y1_ref[1] = y[:, 128:]


def _tc2_body(s1_ref, y1_ref, d0_ref, d1_ref, b1_ref, w2_ref, y2_ref):
    dinv = _dinv_block(d0_ref, d1_ref)
    b1 = b1_ref[...]
    h0 = jnp.maximum(dinv * (s1_ref[0] + y1_ref[0]) + b1[:, :128], 0.0)
    h1 = jnp.maximum(dinv * (s1_ref[1] + y1_ref[1]) + b1[:, 128:], 0.0)
    h = jnp.concatenate([h0, h1], axis=1)
    y2_ref[...] = jnp.dot(h, w2_ref[...], preferred_element_type=F32) * dinv


def _tc3_body(s2_ref, y2_ref, d0_ref, d1_ref, b2_ref,
              f1w_ref, f1b_ref, f2w_ref, f2b_ref, o_ref):
    dinv = _dinv_block(d0_ref, d1_ref)
    z = jnp.maximum(dinv * (s2_ref[0] + s2_ref[1] + y2_ref[...])
                    + b2_ref[...], 0.0)
    t = jnp.dot(z, f1w_ref[...], preferred_element_type=F32) + f1b_ref[...]
    p = jnp.where(t > 0.0, t, jnp.exp(t) - 1.0)
    o_ref[...] = jnp.dot(p, f2w_ref[...], preferred_element_type=F32) \
        + f2b_ref[...]


def _row_spec(shape):
    nd = len(shape)
    if nd == 2:
        return pl.BlockSpec((RB, shape[1]), lambda i: (i, 0))
    return pl.BlockSpec((shape[0], RB, shape[2]), lambda i: (0, i, 0))


def _full_spec(shape):
    return pl.BlockSpec(shape, lambda i: (0,) * len(shape))


def _tc_call(body, ins_row, ins_full, out_shapes):
    in_specs = [_row_spec(a.shape) for a in ins_row] + \
               [_full_spec(a.shape) for a in ins_full]
    out_specs = [_row_spec(s.shape) for s in out_shapes]
    outs = pl.pallas_call(
        body,
        grid=(GRID,),
        in_specs=in_specs,
        out_specs=out_specs if len(out_shapes) > 1 else out_specs[0],
        out_shape=out_shapes if len(out_shapes) > 1 else out_shapes[0],
    )(*ins_row, *ins_full)
    return outs


def kernel(x, edge_index, W1, b1, W2, b2, fc1_W, fc1_b, fc2_W, fc2_b):
    ei = edge_index.astype(I32)
    src, dst = ei[0], ei[1]
    pad = EP - E
    fk = jnp.arange(pad, dtype=I32)
    fake_src = (fk * 131) % N
    fake_dst = N + (fk % 32)
    srcp = jnp.concatenate([src, fake_src]).reshape(ROWS, K)
    dstp = jnp.concatenate([dst, fake_dst]).reshape(ROWS, K)

    d0, d1 = _deg_kernel(dstp)
    d0 = d0.reshape(DTROWS, 1)
    d1 = d1.reshape(DTROWS, 1)

    sds = jax.ShapeDtypeStruct
    y1 = _tc_call(
        _tc1_body, [x, d0, d1], [W1],
        [sds((NC, N, 128), F32)])

    s1 = _conv1_kernel(srcp, dstp, y1)

    y2 = _tc_call(
        _tc2_body, [s1, y1, d0, d1],
        [b1.reshape(1, D_HID), W2],
        [sds((N, 128), F32)])

    s2 = _conv2_kernel(srcp, dstp, y2)

    out = _tc_call(
        _tc3_body, [s2, y2, d0, d1],
        [b2.reshape(1, 128), fc1_W, fc1_b.reshape(1, 128),
         fc2_W, fc2_b.reshape(1, 128)],
        [sds((N, D_OUT), F32)])
    return out


# speedup vs baseline: 29.3334x; 1.0177x over previous
import functools

import jax
import jax.numpy as jnp
from jax import lax
from jax.experimental import pallas as pl
from jax.experimental.pallas import tpu as pltpu
from jax.experimental.pallas import tpu_sc as plsc

F32 = jnp.float32
I32 = jnp.int32

N = 10000
E = 320000
D_HID = 256
D_OUT = 128

NC, NS = 2, 16
NW = NC * NS
K = 64
WPW = 162
CWPW = 324
EP = NW * WPW * K
ROWS = EP // K
TROWS = 10032
RPT = TROWS // NS
NWB = 11
WBR = RPT // NWB
DTROWS = 10240
DRPT = DTROWS // NS
NBUF = 6
DBUF = 9

_mesh = plsc.VectorSubcoreMesh(core_axis_name="c", subcore_axis_name="s")


def _deg_body(dst_hbm, deg0_out, deg1_out, deg_sh, dst_v, ones_v, wb_v,
              dsem):
    cid = lax.axis_index("c")
    sid = lax.axis_index("s")
    wid = cid * NS + sid

    for l in range(K // 16):
        ones_v[pl.ds(l * 16, 16)] = jnp.full((16,), 1.0, F32)

    def zr(i, carry):
        wb_v[pl.ds(i * 16, 16)] = jnp.zeros((16,), F32)
        return carry
    lax.fori_loop(0, DRPT // 16, zr, 0)

    pltpu.sync_copy(wb_v, deg_sh.at[pl.ds(sid * DRPT, DRPT)])
    plsc.subcore_barrier()

    pltpu.sync_copy(dst_hbm.at[pl.ds(wid * WPW, WPW)], dst_v)

    def step(g, carry):
        for b in range(DBUF):
            j = g * DBUF + b
            pltpu.async_copy(ones_v, deg_sh.at[dst_v.at[j]], dsem, add=True)
        for b in range(DBUF):
            j = g * DBUF + b
            pltpu.make_async_copy(ones_v, deg_sh.at[dst_v.at[j]],
                                  dsem).wait()
        return carry
    lax.fori_loop(0, WPW // DBUF, step, 0)

    plsc.subcore_barrier()
    pltpu.sync_copy(deg_sh.at[pl.ds(sid * DRPT, DRPT)], wb_v)

    @pl.when(cid == 0)
    def _wb0():
        pltpu.sync_copy(wb_v, deg0_out.at[pl.ds(sid * DRPT, DRPT)])

    @pl.when(cid == 1)
    def _wb1():
        pltpu.sync_copy(wb_v, deg1_out.at[pl.ds(sid * DRPT, DRPT)])


_deg_kernel = pl.kernel(
    _deg_body,
    out_type=[jax.ShapeDtypeStruct((DTROWS,), F32)] * 2,
    mesh=_mesh,
    compiler_params=pltpu.CompilerParams(use_tc_tiling_on_sc=False),
    scratch_types=[
        pltpu.VMEM_SHARED((DTROWS,), F32),
        pltpu.VMEM((WPW, K), I32),
        pltpu.VMEM((K,), F32),
        pltpu.VMEM((DRPT,), F32),
        pltpu.SemaphoreType.DMA,
    ],
)


def _conv_body(split_edges, src_hbm, dst_hbm, y_hbm, out_hbm,
               src_v, dst_v, acc_sh, rbuf, gsem, ssem, isem):
    cid = lax.axis_index("c")
    sid = lax.axis_index("s")
    if split_edges:
        nwin = WPW
        slab = (cid * NS + sid) * WPW
        yv = y_hbm
    else:
        nwin = CWPW
        slab = sid * CWPW
        yv = y_hbm.at[cid]
    ngrp = nwin // NBUF
    ov = out_hbm.at[cid]

    pltpu.async_copy(src_hbm.at[pl.ds(slab, NBUF)], src_v.at[0], isem.at[0])
    pltpu.async_copy(dst_hbm.at[pl.ds(slab, NBUF)], dst_v.at[0], isem.at[0])

    def zrow(i, carry):
        for l in range(8):
            rbuf[0, i, pl.ds(l * 16, 16)] = jnp.zeros((16,), F32)
        return carry
    lax.fori_loop(0, WBR, zrow, 0)

    def zcp(q, carry):
        pltpu.async_copy(rbuf.at[0, pl.ds(0, WBR)],
                         acc_sh.at[pl.ds(sid * RPT + q * WBR, WBR), :],
                         ssem.at[0])
        return carry
    lax.fori_loop(0, NWB, zcp, 0)

    def zdr(q, carry):
        pltpu.make_async_copy(
            rbuf.at[0, pl.ds(0, WBR)],
            acc_sh.at[pl.ds(sid * RPT, WBR), :], ssem.at[0]).wait()
        return carry
    lax.fori_loop(0, NWB, zdr, 0)
    plsc.subcore_barrier()

    pltpu.make_async_copy(src_hbm.at[pl.ds(slab, NBUF)], src_v.at[0],
                          isem.at[0]).wait()
    pltpu.make_async_copy(dst_hbm.at[pl.ds(slab, NBUF)], dst_v.at[0],
                          isem.at[0]).wait()
    pltpu.async_copy(src_hbm.at[pl.ds(slab + NBUF, NBUF)], src_v.at[1],
                     isem.at[1])
    pltpu.async_copy(dst_hbm.at[pl.ds(slab + NBUF, NBUF)], dst_v.at[1],
                     isem.at[1])

    def prime(b, carry):
        pltpu.async_copy(yv.at[src_v.at[0, b]], rbuf.at[b], gsem.at[b])
        return carry
    lax.fori_loop(0, NBUF, prime, 0)

    def step(g, carry):
        gb = lax.rem(g, 2)
        nb = lax.rem(g + 1, 2)

        def ph1(b, carry):
            pltpu.make_async_copy(
                yv.at[src_v.at[gb, b]], rbuf.at[b], gsem.at[b]).wait()
            pltpu.async_copy(
                rbuf.at[b], acc_sh.at[dst_v.at[gb, b]], ssem.at[b],
                add=True)
            return carry
        lax.fori_loop(0, NBUF, ph1, 0)

        @pl.when(g + 1 < ngrp)
        def _wi():
            pltpu.make_async_copy(
                src_hbm.at[pl.ds(slab, NBUF)], src_v.at[nb],
                isem.at[nb]).wait()
            pltpu.make_async_copy(
                dst_hbm.at[pl.ds(slab, NBUF)], dst_v.at[nb],
                isem.at[nb]).wait()

        def ph2(b, carry):
            pltpu.make_async_copy(
                rbuf.at[b], acc_sh.at[dst_v.at[gb, b]], ssem.at[b]).wait()

            @pl.when(g + 1 < ngrp)
            def _issue_next():
                pltpu.async_copy(
                    yv.at[src_v.at[nb, b]], rbuf.at[b], gsem.at[b])
            return carry
        lax.fori_loop(0, NBUF, ph2, 0)

        @pl.when(g + 2 < ngrp)
        def _pf():
            base = slab + (g + 2) * NBUF
            pltpu.async_copy(src_hbm.at[pl.ds(base, NBUF)], src_v.at[gb],
                             isem.at[gb])
            pltpu.async_copy(dst_hbm.at[pl.ds(base, NBUF)], dst_v.at[gb],
                             isem.at[gb])
        return carry
    lax.fori_loop(0, ngrp, step, 0)

    plsc.subcore_barrier()
    def wprime(q, carry):
        pltpu.async_copy(
            acc_sh.at[pl.ds(sid * RPT + q * WBR, WBR), :],
            rbuf.at[lax.rem(q, NBUF), pl.ds(0, WBR)],
            gsem.at[lax.rem(q, NBUF)])
        return carry
    lax.fori_loop(0, NBUF, wprime, 0)

    def wchain(q, carry):
        b = lax.rem(q, NBUF)
        pltpu.make_async_copy(
            acc_sh.at[pl.ds(sid * RPT, WBR), :],
            rbuf.at[b, pl.ds(0, WBR)], gsem.at[b]).wait()
        pltpu.async_copy(
            rbuf.at[b, pl.ds(0, WBR)],
            ov.at[pl.ds(sid * RPT + q * WBR, WBR), :], ssem.at[b])

        @pl.when(q + NBUF < NWB)
        def _next_rd():
            pltpu.make_async_copy(
                rbuf.at[b, pl.ds(0, WBR)],
                ov.at[pl.ds(sid * RPT, WBR), :], ssem.at[b]).wait()
            pltpu.async_copy(
                acc_sh.at[pl.ds(sid * RPT + (q + NBUF) * WBR, WBR), :],
                rbuf.at[b, pl.ds(0, WBR)], gsem.at[b])
        return carry
    lax.fori_loop(0, NWB, wchain, 0)

    def wdrain(q, carry):
        b = lax.rem(q, NBUF)
        pltpu.make_async_copy(
            rbuf.at[b, pl.ds(0, WBR)],
            ov.at[pl.ds(sid * RPT, WBR), :], ssem.at[b]).wait()
        return carry
    lax.fori_loop(NWB - NBUF, NWB, wdrain, 0)


def _make_conv(split_edges, y_shape):
    return pl.kernel(
        functools.partial(_conv_body, split_edges),
        out_type=jax.ShapeDtypeStruct((NC, TROWS, 128), F32),
        mesh=_mesh,
        compiler_params=pltpu.CompilerParams(use_tc_tiling_on_sc=False),
        scratch_types=[
            pltpu.VMEM((2, NBUF, K), I32),
            pltpu.VMEM((2, NBUF, K), I32),
            pltpu.VMEM_SHARED((TROWS, 128), F32),
            pltpu.VMEM((NBUF, K, 128), F32),
            pltpu.SemaphoreType.DMA((NBUF,)),
            pltpu.SemaphoreType.DMA((NBUF,)),
            pltpu.SemaphoreType.DMA((2,)),
        ],
    )


_conv1_kernel = _make_conv(False, (NC, N, 128))
_conv2_kernel = _make_conv(True, (N, 128))


RB = 2000
GRID = N // RB


def _dinv_block(d0_ref, d1_ref):
    return lax.rsqrt(d0_ref[...] + d1_ref[...] + 1.0)


def _tc1_body(x_ref, d0_ref, d1_ref, w1_ref, y1_ref):
    dinv = _dinv_block(d0_ref, d1_ref)
    xw = jnp.dot(x_ref[...], w1_ref[...], preferred_element_type=F32)
    y = xw * dinv
    y1_ref[0] = y[:, :128]
    y1_ref[1] = y[:, 128:]


def _tc2_body(s1_ref, y1_ref, d0_ref, d1_ref, b1_ref, w2_ref, y2_ref):
    dinv = _dinv_block(d0_ref, d1_ref)
    b1 = b1_ref[...]
    h0 = jnp.maximum(dinv * (s1_ref[0] + y1_ref[0]) + b1[:, :128], 0.0)
    h1 = jnp.maximum(dinv * (s1_ref[1] + y1_ref[1]) + b1[:, 128:], 0.0)
    h = jnp.concatenate([h0, h1], axis=1)
    y2_ref[...] = jnp.dot(h, w2_ref[...], preferred_element_type=F32) * dinv


def _tc3_body(s2_ref, y2_ref, d0_ref, d1_ref, b2_ref,
              f1w_ref, f1b_ref, f2w_ref, f2b_ref, o_ref):
    dinv = _dinv_block(d0_ref, d1_ref)
    z = jnp.maximum(dinv * (s2_ref[0] + s2_ref[1] + y2_ref[...])
                    + b2_ref[...], 0.0)
    t = jnp.dot(z, f1w_ref[...], preferred_element_type=F32) + f1b_ref[...]
    p = jnp.where(t > 0.0, t, jnp.exp(t) - 1.0)
    o_ref[...] = jnp.dot(p, f2w_ref[...], preferred_element_type=F32) \
        + f2b_ref[...]


def _row_spec(shape):
    nd = len(shape)
    if nd == 2:
        return pl.BlockSpec((RB, shape[1]), lambda i: (i, 0))
    return pl.BlockSpec((shape[0], RB, shape[2]), lambda i: (0, i, 0))


def _full_spec(shape):
    return pl.BlockSpec(shape, lambda i: (0,) * len(shape))


def _tc_call(body, ins_row, ins_full, out_shapes):
    in_specs = [_row_spec(a.shape) for a in ins_row] + \
               [_full_spec(a.shape) for a in ins_full]
    out_specs = [_row_spec(s.shape) for s in out_shapes]
    outs = pl.pallas_call(
        body,
        grid=(GRID,),
        in_specs=in_specs,
        out_specs=out_specs if len(out_shapes) > 1 else out_specs[0],
        out_shape=out_shapes if len(out_shapes) > 1 else out_shapes[0],
    )(*ins_row, *ins_full)
    return outs


def kernel(x, edge_index, W1, b1, W2, b2, fc1_W, fc1_b, fc2_W, fc2_b):
    ei = edge_index.astype(I32)
    src, dst = ei[0], ei[1]
    pad = EP - E
    fk = jnp.arange(pad, dtype=I32)
    fake_src = (fk * 131) % N
    fake_dst = N + (fk % 32)
    srcp = jnp.concatenate([src, fake_src]).reshape(ROWS, K)
    dstp = jnp.concatenate([dst, fake_dst]).reshape(ROWS, K)

    d0, d1 = _deg_kernel(dstp)
    d0 = d0.reshape(DTROWS, 1)
    d1 = d1.reshape(DTROWS, 1)

    sds = jax.ShapeDtypeStruct
    y1 = _tc_call(
        _tc1_body, [x, d0, d1], [W1],
        [sds((NC, N, 128), F32)])

    s1 = _conv1_kernel(srcp, dstp, y1)

    y2 = _tc_call(
        _tc2_body, [s1, y1, d0, d1],
        [b1.reshape(1, D_HID), W2],
        [sds((N, 128), F32)])

    s2 = _conv2_kernel(srcp, dstp, y2)

    out = _tc_call(
        _tc3_body, [s2, y2, d0, d1],
        [b2.reshape(1, 128), fc1_W, fc1_b.reshape(1, 128),
         fc2_W, fc2_b.reshape(1, 128)],
        [sds((N, D_OUT), F32)])
    return out
